# baseline (speedup 1.0000x reference)
import functools

import jax
import jax.numpy as jnp
from jax import lax
from jax.experimental import pallas as pl
from jax.experimental.pallas import tpu as pltpu
from jax.experimental.pallas import tpu_sc as plsc

N = 10000
E = 320000
D = 128
FP = 2048
G = 512

NC = 2
NS = 16
NW = NC * NS

K = 125
PER_TILE = E // NW
CH = PER_TILE // K
GC = 2
NG = CH // GC
ROWS_MAIN = 624
ROWS_TAIL = N - NS * ROWS_MAIN


def _sc_agg_body(table, src3g, dst3g, out, acc, src_g, dst_g, buf0, buf1,
                 buf2, g0s, g1s, g2s, s0s, s1s, s2s, i0s, i1s, i2s):
    c = lax.axis_index("c")
    s = lax.axis_index("s")
    wid = c * NS + s
    grow = wid * NG
    bufs = (buf0, buf1, buf2)
    gsems = (g0s, g1s, g2s)
    ssems = (s0s, s1s, s2s)
    isems = (i0s, i1s, i2s)

    def load_grp(g, slot):
        pltpu.async_copy(src3g.at[grow + g], src_g.at[slot], isems[slot])
        pltpu.async_copy(dst3g.at[grow + g], dst_g.at[slot], isems[slot])

    def wait_grp(slot):
        pltpu.make_async_copy(src3g.at[0], src_g.at[slot],
                              isems[slot]).wait()
        pltpu.make_async_copy(dst3g.at[0], dst_g.at[slot],
                              isems[slot]).wait()

    load_grp(0, 0)
    load_grp(1, 1)
    wait_grp(0)
    pltpu.async_copy(table.at[src_g.at[0, 0]], buf0, g0s)
    pltpu.async_copy(table.at[src_g.at[0, 1]], buf1, g1s)

    zero = jnp.zeros((16,), jnp.float32)

    def zrow(r, carry):
        for cc in range(D // 16):
            buf2[r, pl.ds(cc * 16, 16)] = zero
        return carry

    lax.fori_loop(0, K, zrow, 0)
    base_row = s * ROWS_MAIN
    for kk in range(ROWS_MAIN // K):
        pltpu.async_copy(buf2, acc.at[pl.ds(base_row + kk * K, K)], s0s)
    rem = ROWS_MAIN % K
    if rem:
        pltpu.async_copy(
            buf2.at[pl.ds(0, rem)],
            acc.at[pl.ds(base_row + (ROWS_MAIN // K) * K, rem)], s0s)

    @pl.when(s == NS - 1)
    def _():
        pltpu.async_copy(buf2.at[pl.ds(0, ROWS_TAIL)],
                         acc.at[pl.ds(NS * ROWS_MAIN, ROWS_TAIL)], s0s)

    for kk in range(ROWS_MAIN // K):
        pltpu.make_async_copy(buf2, acc.at[pl.ds(base_row, K)], s0s).wait()
    if rem:
        pltpu.make_async_copy(buf2.at[pl.ds(0, rem)],
                              acc.at[pl.ds(base_row, rem)], s0s).wait()

    @pl.when(s == NS - 1)
    def _():
        pltpu.make_async_copy(buf2.at[pl.ds(0, ROWS_TAIL)],
                              acc.at[pl.ds(base_row, ROWS_TAIL)],
                              s0s).wait()

    plsc.subcore_barrier()

    for j in range(CH):
        b = j % 3
        g = j // GC
        slot = g % 3
        row = j % GC
        pltpu.make_async_copy(table.at[src_g.at[0, 0]], bufs[b],
                              gsems[b]).wait()
        pltpu.async_copy(bufs[b], acc.at[dst_g.at[slot, row]], ssems[b],
                         add=True)
        jn = j + 2
        if jn < CH:
            if jn >= 3:
                pltpu.make_async_copy(bufs[jn % 3],
                                      acc.at[dst_g.at[0, 0]],
                                      ssems[jn % 3]).wait()
            slot_n = (g + 1) % 3
            row_n = jn % GC
            if row_n == 0:
                wait_grp(slot_n)
            pltpu.async_copy(table.at[src_g.at[slot_n, row_n]],
                             bufs[jn % 3], gsems[jn % 3])
        if row == 0 and g + 2 < NG:
            load_grp(g + 2, (g + 2) % 3)
    for j in range(CH - 3, CH):
        pltpu.make_async_copy(bufs[j % 3], acc.at[dst_g.at[0, 0]],
                              ssems[j % 3]).wait()

    plsc.subcore_barrier()
    pltpu.sync_copy(acc.at[pl.ds(base_row, ROWS_MAIN)],
                    out.at[c, pl.ds(base_row, ROWS_MAIN)])

    @pl.when(s == NS - 1)
    def _():
        pltpu.sync_copy(acc.at[pl.ds(NS * ROWS_MAIN, ROWS_TAIL)],
                        out.at[c, pl.ds(NS * ROWS_MAIN, ROWS_TAIL)])


_sc_agg = functools.partial(
    pl.kernel,
    out_type=jax.ShapeDtypeStruct((NC, N, D), jnp.float32),
    cost_estimate=pl.CostEstimate(flops=85_000_000, transcendentals=0,
                                  bytes_accessed=200_000_000),
    mesh=plsc.VectorSubcoreMesh(core_axis_name="c", subcore_axis_name="s",
                                num_cores=NC, num_subcores=NS),
    scratch_types=[
        pltpu.VMEM_SHARED((N, D), jnp.float32),
        pltpu.VMEM((3, GC, K), jnp.int32),
        pltpu.VMEM((3, GC, K), jnp.int32),
        pltpu.VMEM((K, D), jnp.float32),
        pltpu.VMEM((K, D), jnp.float32),
        pltpu.VMEM((K, D), jnp.float32),
    ] + [pltpu.SemaphoreType.DMA] * 9,
)(_sc_agg_body)


def _tc_layer_body(p_ref, x_ref, w_ref, b_ref, o_ref):
    sm = p_ref[0] + p_ref[1] + x_ref[...]
    z = jnp.dot(sm, w_ref[...], preferred_element_type=jnp.float32)
    o_ref[...] = jax.nn.sigmoid(z + b_ref[...])


_LAYER_BLK = 2000


def _tc_layer(p, x, wt, b):
    nb = N // _LAYER_BLK
    return pl.pallas_call(
        _tc_layer_body,
        grid=(nb,),
        in_specs=[
            pl.BlockSpec((NC, _LAYER_BLK, D), lambda i: (0, i, 0)),
            pl.BlockSpec((_LAYER_BLK, D), lambda i: (i, 0)),
            pl.BlockSpec((D, D), lambda i: (0, 0)),
            pl.BlockSpec((1, D), lambda i: (0, 0)),
        ],
        out_specs=pl.BlockSpec((_LAYER_BLK, D), lambda i: (i, 0)),
        out_shape=jax.ShapeDtypeStruct((N, D), jnp.float32),
    )(p, x, wt, b)


_FIN_BLK = 400


SPAN = 64


def _fp_contribs(h1_ref, h2_ref, w1_ref, b1_ref, w2_ref, b2_ref, bt_ref,
                 o_ref):
    l1 = jnp.dot(h1_ref[...].astype(jnp.bfloat16), w1_ref[...],
                 preferred_element_type=jnp.float32) + b1_ref[...]
    e1 = jnp.exp(l1)
    eb1 = e1.astype(jnp.bfloat16)
    inv1 = 1.0 / jnp.sum(e1, axis=1, keepdims=True)
    l2 = jnp.dot(h2_ref[...].astype(jnp.bfloat16), w2_ref[...],
                 preferred_element_type=jnp.float32) + b2_ref[...]
    e2 = jnp.exp(l2)
    eb2 = e2.astype(jnp.bfloat16)
    inv2 = 1.0 / jnp.sum(e2, axis=1, keepdims=True)
    gid = bt_ref[0, 0, :]
    g0 = jnp.minimum((jnp.min(gid) // 8) * 8, G - SPAN)
    fast = (jnp.max(gid) - g0) < SPAN

    @pl.when(fast)
    def _():
        cmp = (gid - g0)[:, None] == lax.broadcasted_iota(
            jnp.int32, (_FIN_BLK, SPAN), 1)
        oh1 = jnp.where(cmp, inv1, 0.0).astype(jnp.bfloat16)
        oh2 = jnp.where(cmp, inv2, 0.0).astype(jnp.bfloat16)
        contrib = (
            lax.dot_general(oh1, eb1, (((0,), (0,)), ((), ())),
                            preferred_element_type=jnp.float32) +
            lax.dot_general(oh2, eb2, (((0,), (0,)), ((), ())),
                            preferred_element_type=jnp.float32))
        o_ref[pl.ds(g0, SPAN), :] += contrib

    @pl.when(jnp.logical_not(fast))
    def _():
        cmp = gid[:, None] == lax.broadcasted_iota(
            jnp.int32, (_FIN_BLK, G), 1)
        oh1 = jnp.where(cmp, inv1, 0.0).astype(jnp.bfloat16)
        oh2 = jnp.where(cmp, inv2, 0.0).astype(jnp.bfloat16)
        contrib = (
            lax.dot_general(oh1, eb1, (((0,), (0,)), ((), ())),
                            preferred_element_type=jnp.float32) +
            lax.dot_general(oh2, eb2, (((0,), (0,)), ((), ())),
                            preferred_element_type=jnp.float32))
        o_ref[...] += contrib


def _tc_fp_body(p_ref, h1_ref, hw_ref, hb_ref, w1_ref, b1_ref, w2_ref,
                b2_ref, bt_ref, o_ref, h2_scr):
    @pl.when(pl.program_id(0) == 0)
    def _():
        o_ref[...] = jnp.zeros((G, FP), jnp.float32)

    sm = p_ref[0] + p_ref[1] + h1_ref[...]
    z = jnp.dot(sm, hw_ref[...], preferred_element_type=jnp.float32)
    h2_scr[...] = jax.nn.sigmoid(z + hb_ref[...])
    _fp_contribs(h1_ref, h2_scr, w1_ref, b1_ref, w2_ref, b2_ref, bt_ref,
                 o_ref)


def _tc_fp(p2, h1, hwt, hb, w1t, b1, w2t, b2, batch3d):
    return pl.pallas_call(
        _tc_fp_body,
        grid=(N // _FIN_BLK,),
        in_specs=[
            pl.BlockSpec((NC, _FIN_BLK, D), lambda i: (0, i, 0)),
            pl.BlockSpec((_FIN_BLK, D), lambda i: (i, 0)),
            pl.BlockSpec((D, D), lambda i: (0, 0)),
            pl.BlockSpec((1, D), lambda i: (0, 0)),
            pl.BlockSpec((D, FP), lambda i: (0, 0)),
            pl.BlockSpec((1, FP), lambda i: (0, 0)),
            pl.BlockSpec((D, FP), lambda i: (0, 0)),
            pl.BlockSpec((1, FP), lambda i: (0, 0)),
            pl.BlockSpec((1, 1, _FIN_BLK), lambda i: (i, 0, 0)),
        ],
        out_specs=pl.BlockSpec((G, FP), lambda i: (0, 0)),
        out_shape=jax.ShapeDtypeStruct((G, FP), jnp.float32),
        scratch_shapes=[pltpu.VMEM((_FIN_BLK, D), jnp.float32)],
    )(p2, h1, hwt, hb, w1t, b1, w2t, b2, batch3d)


def kernel(x, edge_index, batch, H1_w, H1_b, W1_w, W1_b, H2_w, H2_b, W2_w,
           W2_b):
    src3d = edge_index[0].reshape(E // (K * GC), GC, K)
    dst3d = edge_index[1].reshape(E // (K * GC), GC, K)
    batch3d = batch.reshape(N // _FIN_BLK, 1, _FIN_BLK)
    w1t = W1_w.T.astype(jnp.bfloat16)
    w2t = W2_w.T.astype(jnp.bfloat16)

    p1 = _sc_agg(x, src3d, dst3d)
    h1 = _tc_layer(p1, x, H1_w.T, H1_b.reshape(1, D))
    p2 = _sc_agg(h1, src3d, dst3d)
    return _tc_fp(p2, h1, H2_w.T, H2_b.reshape(1, D),
                  w1t, W1_b.reshape(1, FP), w2t, W2_b.reshape(1, FP),
                  batch3d)


# speedup vs baseline: 1.0695x; 1.0695x over previous
import functools

import jax
import jax.numpy as jnp
from jax import lax
from jax.experimental import pallas as pl
from jax.experimental.pallas import tpu as pltpu
from jax.experimental.pallas import tpu_sc as plsc

N = 10000
E = 320000
D = 128
FP = 2048
G = 512

NC = 2
NS = 16
NW = NC * NS

K = 125
PER_TILE = E // NW
CH = PER_TILE // K
GC = 2
NG = CH // GC
ROWS_MAIN = 624
ROWS_TAIL = N - NS * ROWS_MAIN


def _sc_agg_body(table, src3g, dst3g, out, acc, src_g, dst_g, buf0, buf1,
                 buf2, g0s, g1s, g2s, s0s, s1s, s2s, i0s, i1s, i2s):
    c = lax.axis_index("c")
    s = lax.axis_index("s")
    wid = c * NS + s
    grow = wid * NG
    bufs = (buf0, buf1, buf2)
    gsems = (g0s, g1s, g2s)
    ssems = (s0s, s1s, s2s)
    isems = (i0s, i1s, i2s)

    def load_grp(g, slot):
        pltpu.async_copy(src3g.at[grow + g], src_g.at[slot], isems[slot])
        pltpu.async_copy(dst3g.at[grow + g], dst_g.at[slot], isems[slot])

    def wait_grp(slot):
        pltpu.make_async_copy(src3g.at[0], src_g.at[slot],
                              isems[slot]).wait()
        pltpu.make_async_copy(dst3g.at[0], dst_g.at[slot],
                              isems[slot]).wait()

    load_grp(0, 0)
    load_grp(1, 1)
    wait_grp(0)
    pltpu.async_copy(table.at[src_g.at[0, 0]], buf0, g0s)
    pltpu.async_copy(table.at[src_g.at[0, 1]], buf1, g1s)

    zero = jnp.zeros((16,), jnp.float32)

    def zrow(r, carry):
        for cc in range(D // 16):
            buf2[r, pl.ds(cc * 16, 16)] = zero
        return carry

    lax.fori_loop(0, K, zrow, 0)
    base_row = s * ROWS_MAIN
    for kk in range(ROWS_MAIN // K):
        pltpu.async_copy(buf2, acc.at[pl.ds(base_row + kk * K, K)], s0s)
    rem = ROWS_MAIN % K
    if rem:
        pltpu.async_copy(
            buf2.at[pl.ds(0, rem)],
            acc.at[pl.ds(base_row + (ROWS_MAIN // K) * K, rem)], s0s)

    @pl.when(s == NS - 1)
    def _():
        pltpu.async_copy(buf2.at[pl.ds(0, ROWS_TAIL)],
                         acc.at[pl.ds(NS * ROWS_MAIN, ROWS_TAIL)], s0s)

    for kk in range(ROWS_MAIN // K):
        pltpu.make_async_copy(buf2, acc.at[pl.ds(base_row, K)], s0s).wait()
    if rem:
        pltpu.make_async_copy(buf2.at[pl.ds(0, rem)],
                              acc.at[pl.ds(base_row, rem)], s0s).wait()

    @pl.when(s == NS - 1)
    def _():
        pltpu.make_async_copy(buf2.at[pl.ds(0, ROWS_TAIL)],
                              acc.at[pl.ds(base_row, ROWS_TAIL)],
                              s0s).wait()

    plsc.subcore_barrier()

    for j in range(CH):
        b = j % 3
        g = j // GC
        slot = g % 3
        row = j % GC
        pltpu.make_async_copy(table.at[src_g.at[0, 0]], bufs[b],
                              gsems[b]).wait()
        pltpu.async_copy(bufs[b], acc.at[dst_g.at[slot, row]], ssems[b],
                         add=True)
        jn = j + 2
        if jn < CH:
            if jn >= 3:
                pltpu.make_async_copy(bufs[jn % 3],
                                      acc.at[dst_g.at[0, 0]],
                                      ssems[jn % 3]).wait()
            slot_n = (g + 1) % 3
            row_n = jn % GC
            if row_n == 0:
                wait_grp(slot_n)
            pltpu.async_copy(table.at[src_g.at[slot_n, row_n]],
                             bufs[jn % 3], gsems[jn % 3])
        if row == 0 and g + 2 < NG:
            load_grp(g + 2, (g + 2) % 3)
    for j in range(CH - 3, CH):
        pltpu.make_async_copy(bufs[j % 3], acc.at[dst_g.at[0, 0]],
                              ssems[j % 3]).wait()

    plsc.subcore_barrier()
    pltpu.sync_copy(acc.at[pl.ds(base_row, ROWS_MAIN)],
                    out.at[c, pl.ds(base_row, ROWS_MAIN)])

    @pl.when(s == NS - 1)
    def _():
        pltpu.sync_copy(acc.at[pl.ds(NS * ROWS_MAIN, ROWS_TAIL)],
                        out.at[c, pl.ds(NS * ROWS_MAIN, ROWS_TAIL)])


_sc_agg = functools.partial(
    pl.kernel,
    out_type=jax.ShapeDtypeStruct((NC, N, D), jnp.float32),
    cost_estimate=pl.CostEstimate(flops=85_000_000, transcendentals=0,
                                  bytes_accessed=200_000_000),
    mesh=plsc.VectorSubcoreMesh(core_axis_name="c", subcore_axis_name="s",
                                num_cores=NC, num_subcores=NS),
    scratch_types=[
        pltpu.VMEM_SHARED((N, D), jnp.float32),
        pltpu.VMEM((3, GC, K), jnp.int32),
        pltpu.VMEM((3, GC, K), jnp.int32),
        pltpu.VMEM((K, D), jnp.float32),
        pltpu.VMEM((K, D), jnp.float32),
        pltpu.VMEM((K, D), jnp.float32),
    ] + [pltpu.SemaphoreType.DMA] * 9,
)(_sc_agg_body)


def _tc_layer_body(p_ref, x_ref, w_ref, b_ref, o_ref):
    sm = p_ref[0] + p_ref[1] + x_ref[...]
    z = jnp.dot(sm, w_ref[...], preferred_element_type=jnp.float32)
    o_ref[...] = jax.nn.sigmoid(z + b_ref[...])


_LAYER_BLK = 2000


def _tc_layer(p, x, wt, b):
    nb = N // _LAYER_BLK
    return pl.pallas_call(
        _tc_layer_body,
        grid=(nb,),
        in_specs=[
            pl.BlockSpec((NC, _LAYER_BLK, D), lambda i: (0, i, 0)),
            pl.BlockSpec((_LAYER_BLK, D), lambda i: (i, 0)),
            pl.BlockSpec((D, D), lambda i: (0, 0)),
            pl.BlockSpec((1, D), lambda i: (0, 0)),
        ],
        out_specs=pl.BlockSpec((_LAYER_BLK, D), lambda i: (i, 0)),
        out_shape=jax.ShapeDtypeStruct((N, D), jnp.float32),
    )(p, x, wt, b)


_FIN_BLK = 400


SPAN = 64


def _fp_accum(h_ref, w_ref, b_ref, bt_ref, o_ref):
    logits = jnp.dot(h_ref[...].astype(jnp.bfloat16), w_ref[...],
                     preferred_element_type=jnp.float32) + b_ref[...]
    e = jnp.exp(logits)
    eb = e.astype(jnp.bfloat16)
    inv = 1.0 / jnp.sum(e, axis=1, keepdims=True)
    gid = bt_ref[0, 0, :]
    g0 = jnp.minimum((jnp.min(gid) // 8) * 8, G - SPAN)
    fast = (jnp.max(gid) - g0) < SPAN

    @pl.when(fast)
    def _():
        onehot = (jnp.where(
            (gid - g0)[:, None] == lax.broadcasted_iota(
                jnp.int32, (_FIN_BLK, SPAN), 1),
            inv, 0.0)).astype(jnp.bfloat16)
        contrib = lax.dot_general(onehot, eb, (((0,), (0,)), ((), ())),
                                  preferred_element_type=jnp.float32)
        o_ref[pl.ds(g0, SPAN), :] += contrib

    @pl.when(jnp.logical_not(fast))
    def _():
        onehot = (jnp.where(
            gid[:, None] == lax.broadcasted_iota(
                jnp.int32, (_FIN_BLK, G), 1),
            inv, 0.0)).astype(jnp.bfloat16)
        contrib = lax.dot_general(onehot, eb, (((0,), (0,)), ((), ())),
                                  preferred_element_type=jnp.float32)
        o_ref[...] += contrib


def _tc_fp1_body(h_ref, w_ref, b_ref, bt_ref, o_ref):
    @pl.when(pl.program_id(0) == 0)
    def _():
        o_ref[...] = jnp.zeros((G, FP), jnp.float32)

    _fp_accum(h_ref, w_ref, b_ref, bt_ref, o_ref)


def _tc_fp2_body(acc_ref, p_ref, h1_ref, hw_ref, hb_ref, w_ref, b_ref,
                 bt_ref, o_ref, h2_scr):
    @pl.when(pl.program_id(0) == 0)
    def _():
        o_ref[...] = acc_ref[...]

    sm = p_ref[0] + p_ref[1] + h1_ref[...]
    z = jnp.dot(sm, hw_ref[...], preferred_element_type=jnp.float32)
    h2_scr[...] = jax.nn.sigmoid(z + hb_ref[...])
    _fp_accum(h2_scr, w_ref, b_ref, bt_ref, o_ref)


_FIN_SPECS = [
    pl.BlockSpec((_FIN_BLK, D), lambda i: (i, 0)),
    pl.BlockSpec((D, FP), lambda i: (0, 0)),
    pl.BlockSpec((1, FP), lambda i: (0, 0)),
    pl.BlockSpec((1, 1, _FIN_BLK), lambda i: (i, 0, 0)),
]


def _tc_fp1(h, wt, b, batch3d):
    return pl.pallas_call(
        _tc_fp1_body,
        grid=(N // _FIN_BLK,),
        in_specs=_FIN_SPECS,
        out_specs=pl.BlockSpec((G, FP), lambda i: (0, 0)),
        out_shape=jax.ShapeDtypeStruct((G, FP), jnp.float32),
    )(h, wt, b, batch3d)


def _tc_fp2(acc, p2, h1, hwt, hb, wt, b, batch3d):
    return pl.pallas_call(
        _tc_fp2_body,
        grid=(N // _FIN_BLK,),
        in_specs=[
            pl.BlockSpec((G, FP), lambda i: (0, 0)),
            pl.BlockSpec((NC, _FIN_BLK, D), lambda i: (0, i, 0)),
            pl.BlockSpec((_FIN_BLK, D), lambda i: (i, 0)),
            pl.BlockSpec((D, D), lambda i: (0, 0)),
            pl.BlockSpec((1, D), lambda i: (0, 0)),
        ] + _FIN_SPECS[1:],
        out_specs=pl.BlockSpec((G, FP), lambda i: (0, 0)),
        out_shape=jax.ShapeDtypeStruct((G, FP), jnp.float32),
        scratch_shapes=[pltpu.VMEM((_FIN_BLK, D), jnp.float32)],
    )(acc, p2, h1, hwt, hb, wt, b, batch3d)


def kernel(x, edge_index, batch, H1_w, H1_b, W1_w, W1_b, H2_w, H2_b, W2_w,
           W2_b):
    src3d = edge_index[0].reshape(E // (K * GC), GC, K)
    dst3d = edge_index[1].reshape(E // (K * GC), GC, K)
    batch3d = batch.reshape(N // _FIN_BLK, 1, _FIN_BLK)
    w1t = W1_w.T.astype(jnp.bfloat16)
    w2t = W2_w.T.astype(jnp.bfloat16)

    p1 = _sc_agg(x, src3d, dst3d)
    h1 = _tc_layer(p1, x, H1_w.T, H1_b.reshape(1, D))
    acc1 = _tc_fp1(h1, w1t, W1_b.reshape(1, FP), batch3d)
    p2 = _sc_agg(h1, src3d, dst3d)
    return _tc_fp2(acc1, p2, h1, H2_w.T, H2_b.reshape(1, D),
                   w2t, W2_b.reshape(1, FP), batch3d)


# speedup vs baseline: 1.1092x; 1.0370x over previous
import functools

import jax
import jax.numpy as jnp
from jax import lax
from jax.experimental import pallas as pl
from jax.experimental.pallas import tpu as pltpu
from jax.experimental.pallas import tpu_sc as plsc

N = 10000
E = 320000
D = 128
FP = 2048
G = 512

NC = 2
NS = 16
NW = NC * NS

K = 125
PER_TILE = E // NW
CH = PER_TILE // K
GC = 2
NG = CH // GC
ROWS_MAIN = 624
ROWS_TAIL = N - NS * ROWS_MAIN


def _sc_agg_body(table, src3g, dst3g, out, acc, src_g, dst_g, buf0, buf1,
                 buf2, g0s, g1s, g2s, s0s, s1s, s2s, i0s, i1s, i2s):
    c = lax.axis_index("c")
    s = lax.axis_index("s")
    wid = c * NS + s
    grow = wid * NG
    bufs = (buf0, buf1, buf2)
    gsems = (g0s, g1s, g2s)
    ssems = (s0s, s1s, s2s)
    isems = (i0s, i1s, i2s)

    def load_grp(g, slot):
        pltpu.async_copy(src3g.at[grow + g], src_g.at[slot], isems[slot])
        pltpu.async_copy(dst3g.at[grow + g], dst_g.at[slot], isems[slot])

    def wait_grp(slot):
        pltpu.make_async_copy(src3g.at[0], src_g.at[slot],
                              isems[slot]).wait()
        pltpu.make_async_copy(dst3g.at[0], dst_g.at[slot],
                              isems[slot]).wait()

    load_grp(0, 0)
    load_grp(1, 1)
    wait_grp(0)
    pltpu.async_copy(table.at[src_g.at[0, 0]], buf0, g0s)
    pltpu.async_copy(table.at[src_g.at[0, 1]], buf1, g1s)

    zero = jnp.zeros((16,), jnp.float32)

    def zrow(r, carry):
        for cc in range(D // 16):
            buf2[r, pl.ds(cc * 16, 16)] = zero
        return carry

    lax.fori_loop(0, K, zrow, 0)
    base_row = s * ROWS_MAIN
    for kk in range(ROWS_MAIN // K):
        pltpu.async_copy(buf2, acc.at[pl.ds(base_row + kk * K, K)], s0s)
    rem = ROWS_MAIN % K
    if rem:
        pltpu.async_copy(
            buf2.at[pl.ds(0, rem)],
            acc.at[pl.ds(base_row + (ROWS_MAIN // K) * K, rem)], s0s)

    @pl.when(s == NS - 1)
    def _():
        pltpu.async_copy(buf2.at[pl.ds(0, ROWS_TAIL)],
                         acc.at[pl.ds(NS * ROWS_MAIN, ROWS_TAIL)], s0s)

    for kk in range(ROWS_MAIN // K):
        pltpu.make_async_copy(buf2, acc.at[pl.ds(base_row, K)], s0s).wait()
    if rem:
        pltpu.make_async_copy(buf2.at[pl.ds(0, rem)],
                              acc.at[pl.ds(base_row, rem)], s0s).wait()

    @pl.when(s == NS - 1)
    def _():
        pltpu.make_async_copy(buf2.at[pl.ds(0, ROWS_TAIL)],
                              acc.at[pl.ds(base_row, ROWS_TAIL)],
                              s0s).wait()

    plsc.subcore_barrier()

    for j in range(CH):
        b = j % 3
        g = j // GC
        slot = g % 3
        row = j % GC
        pltpu.make_async_copy(table.at[src_g.at[0, 0]], bufs[b],
                              gsems[b]).wait()
        pltpu.async_copy(bufs[b], acc.at[dst_g.at[slot, row]], ssems[b],
                         add=True)
        jn = j + 2
        if jn < CH:
            if jn >= 3:
                pltpu.make_async_copy(bufs[jn % 3],
                                      acc.at[dst_g.at[0, 0]],
                                      ssems[jn % 3]).wait()
            slot_n = (g + 1) % 3
            row_n = jn % GC
            if row_n == 0:
                wait_grp(slot_n)
            pltpu.async_copy(table.at[src_g.at[slot_n, row_n]],
                             bufs[jn % 3], gsems[jn % 3])
        if row == 0 and g + 2 < NG:
            load_grp(g + 2, (g + 2) % 3)
    for j in range(CH - 3, CH):
        pltpu.make_async_copy(bufs[j % 3], acc.at[dst_g.at[0, 0]],
                              ssems[j % 3]).wait()

    plsc.subcore_barrier()
    pltpu.sync_copy(acc.at[pl.ds(base_row, ROWS_MAIN)],
                    out.at[c, pl.ds(base_row, ROWS_MAIN)])

    @pl.when(s == NS - 1)
    def _():
        pltpu.sync_copy(acc.at[pl.ds(NS * ROWS_MAIN, ROWS_TAIL)],
                        out.at[c, pl.ds(NS * ROWS_MAIN, ROWS_TAIL)])


_sc_agg = functools.partial(
    pl.kernel,
    out_type=jax.ShapeDtypeStruct((NC, N, D), jnp.float32),
    cost_estimate=pl.CostEstimate(flops=85_000_000, transcendentals=0,
                                  bytes_accessed=200_000_000),
    mesh=plsc.VectorSubcoreMesh(core_axis_name="c", subcore_axis_name="s",
                                num_cores=NC, num_subcores=NS),
    scratch_types=[
        pltpu.VMEM_SHARED((N, D), jnp.float32),
        pltpu.VMEM((3, GC, K), jnp.int32),
        pltpu.VMEM((3, GC, K), jnp.int32),
        pltpu.VMEM((K, D), jnp.float32),
        pltpu.VMEM((K, D), jnp.float32),
        pltpu.VMEM((K, D), jnp.float32),
    ] + [pltpu.SemaphoreType.DMA] * 9,
)(_sc_agg_body)


def _tc_layer_body(p_ref, x_ref, w_ref, b_ref, o_ref):
    sm = p_ref[0] + p_ref[1] + x_ref[...]
    z = jnp.dot(sm, w_ref[...], preferred_element_type=jnp.float32)
    o_ref[...] = jax.nn.sigmoid(z + b_ref[...])


_LAYER_BLK = 2000


def _tc_layer(p, x, wt, b):
    nb = N // _LAYER_BLK
    return pl.pallas_call(
        _tc_layer_body,
        grid=(nb,),
        in_specs=[
            pl.BlockSpec((NC, _LAYER_BLK, D), lambda i: (0, i, 0)),
            pl.BlockSpec((_LAYER_BLK, D), lambda i: (i, 0)),
            pl.BlockSpec((D, D), lambda i: (0, 0)),
            pl.BlockSpec((1, D), lambda i: (0, 0)),
        ],
        out_specs=pl.BlockSpec((_LAYER_BLK, D), lambda i: (i, 0)),
        out_shape=jax.ShapeDtypeStruct((N, D), jnp.float32),
    )(p, x, wt, b)


_FIN_BLK = 1000


SPAN = 128


def _fp_accum(h_ref, w_ref, b_ref, bt_ref, o_ref):
    logits = jnp.dot(h_ref[...].astype(jnp.bfloat16), w_ref[...],
                     preferred_element_type=jnp.float32) + b_ref[...]
    e = jnp.exp(logits)
    eb = e.astype(jnp.bfloat16)
    inv = 1.0 / jnp.sum(e, axis=1, keepdims=True)
    gid = bt_ref[0, 0, :]
    g0 = jnp.minimum((jnp.min(gid) // 8) * 8, G - SPAN)
    fast = (jnp.max(gid) - g0) < SPAN

    @pl.when(fast)
    def _():
        onehot = (jnp.where(
            (gid - g0)[:, None] == lax.broadcasted_iota(
                jnp.int32, (_FIN_BLK, SPAN), 1),
            inv, 0.0)).astype(jnp.bfloat16)
        contrib = lax.dot_general(onehot, eb, (((0,), (0,)), ((), ())),
                                  preferred_element_type=jnp.float32)
        o_ref[pl.ds(g0, SPAN), :] += contrib

    @pl.when(jnp.logical_not(fast))
    def _():
        onehot = (jnp.where(
            gid[:, None] == lax.broadcasted_iota(
                jnp.int32, (_FIN_BLK, G), 1),
            inv, 0.0)).astype(jnp.bfloat16)
        contrib = lax.dot_general(onehot, eb, (((0,), (0,)), ((), ())),
                                  preferred_element_type=jnp.float32)
        o_ref[...] += contrib


def _tc_fp1_body(h_ref, w_ref, b_ref, bt_ref, o_ref):
    @pl.when(pl.program_id(0) == 0)
    def _():
        o_ref[...] = jnp.zeros((G, FP), jnp.float32)

    _fp_accum(h_ref, w_ref, b_ref, bt_ref, o_ref)


def _tc_fp2_body(acc_ref, p_ref, h1_ref, hw_ref, hb_ref, w_ref, b_ref,
                 bt_ref, o_ref, h2_scr):
    @pl.when(pl.program_id(0) == 0)
    def _():
        o_ref[...] = acc_ref[...]

    sm = p_ref[0] + p_ref[1] + h1_ref[...]
    z = jnp.dot(sm, hw_ref[...], preferred_element_type=jnp.float32)
    h2_scr[...] = jax.nn.sigmoid(z + hb_ref[...])
    _fp_accum(h2_scr, w_ref, b_ref, bt_ref, o_ref)


_FIN_SPECS = [
    pl.BlockSpec((_FIN_BLK, D), lambda i: (i, 0)),
    pl.BlockSpec((D, FP), lambda i: (0, 0)),
    pl.BlockSpec((1, FP), lambda i: (0, 0)),
    pl.BlockSpec((1, 1, _FIN_BLK), lambda i: (i, 0, 0)),
]


def _tc_fp1(h, wt, b, batch3d):
    return pl.pallas_call(
        _tc_fp1_body,
        grid=(N // _FIN_BLK,),
        in_specs=_FIN_SPECS,
        out_specs=pl.BlockSpec((G, FP), lambda i: (0, 0)),
        out_shape=jax.ShapeDtypeStruct((G, FP), jnp.float32),
    )(h, wt, b, batch3d)


def _tc_fp2(acc, p2, h1, hwt, hb, wt, b, batch3d):
    return pl.pallas_call(
        _tc_fp2_body,
        grid=(N // _FIN_BLK,),
        in_specs=[
            pl.BlockSpec((G, FP), lambda i: (0, 0)),
            pl.BlockSpec((NC, _FIN_BLK, D), lambda i: (0, i, 0)),
            pl.BlockSpec((_FIN_BLK, D), lambda i: (i, 0)),
            pl.BlockSpec((D, D), lambda i: (0, 0)),
            pl.BlockSpec((1, D), lambda i: (0, 0)),
        ] + _FIN_SPECS[1:],
        out_specs=pl.BlockSpec((G, FP), lambda i: (0, 0)),
        out_shape=jax.ShapeDtypeStruct((G, FP), jnp.float32),
        scratch_shapes=[pltpu.VMEM((_FIN_BLK, D), jnp.float32)],
    )(acc, p2, h1, hwt, hb, wt, b, batch3d)


def kernel(x, edge_index, batch, H1_w, H1_b, W1_w, W1_b, H2_w, H2_b, W2_w,
           W2_b):
    src3d = edge_index[0].reshape(E // (K * GC), GC, K)
    dst3d = edge_index[1].reshape(E // (K * GC), GC, K)
    batch3d = batch.reshape(N // _FIN_BLK, 1, _FIN_BLK)
    w1t = W1_w.T.astype(jnp.bfloat16)
    w2t = W2_w.T.astype(jnp.bfloat16)

    p1 = _sc_agg(x, src3d, dst3d)
    h1 = _tc_layer(p1, x, H1_w.T, H1_b.reshape(1, D))
    acc1 = _tc_fp1(h1, w1t, W1_b.reshape(1, FP), batch3d)
    p2 = _sc_agg(h1, src3d, dst3d)
    return _tc_fp2(acc1, p2, h1, H2_w.T, H2_b.reshape(1, D),
                   w2t, W2_b.reshape(1, FP), batch3d)
